# Initial kernel scaffold; baseline (speedup 1.0000x reference)
import functools

import jax
import jax.numpy as jnp
from jax import lax
from jax.experimental import pallas as pl
from jax.experimental.pallas import tpu as pltpu
from jax.experimental.pallas import tpu_sc as plsc

SIZE = 1000000
DIM = 64
BATCH = 16384
BAG = 50

NCORE = 2
NSUB = 16
NW = NCORE * NSUB
BPT = BATCH // NW
CB = 8
NCH = BPT // CB
SUB = 4
IPS = CB * BAG // SUB
LANES = 16
DV = DIM // LANES


def _bag_body(idx_hbm, scr_hbm, tbl_hbm, out_hbm, idx_v, scr_v, rows_v, out_v,
              sem):
    wid = lax.axis_index("s") * NCORE + lax.axis_index("c")

    pltpu.sync_copy(idx_hbm.at[wid], idx_v)
    pltpu.sync_copy(scr_hbm.at[wid], scr_v)

    def chunk(g, _):
        for s in range(SUB):
            pltpu.async_copy(
                tbl_hbm.at[idx_v.at[g * SUB + s]],
                rows_v.at[pl.ds(s * IPS, IPS)],
                sem,
            )
        for s in range(SUB):
            pltpu.make_async_copy(
                tbl_hbm.at[idx_v.at[g * SUB + s]],
                rows_v.at[pl.ds(s * IPS, IPS)],
                sem,
            ).wait()

        sbase = g * (CB * BAG)

        def bag(c, _):
            accs = [jnp.zeros((LANES,), jnp.float32) for _ in range(DV)]
            for j in range(BAG):
                r = c * BAG + j
                sc = scr_v[sbase + r]
                for t in range(DV):
                    accs[t] = accs[t] + sc * rows_v[r, pl.ds(t * LANES, LANES)]
            for t in range(DV):
                out_v[c, pl.ds(t * LANES, LANES)] = accs[t]
            return 0

        lax.fori_loop(0, CB, bag, 0)
        pltpu.sync_copy(out_v, out_hbm.at[pl.ds(wid * BPT + g * CB, CB)])
        return 0

    lax.fori_loop(0, NCH, chunk, 0)


@jax.jit
def _bag_call(idx3, scr2, weight):
    mesh = plsc.VectorSubcoreMesh(core_axis_name="c", subcore_axis_name="s")
    return pl.kernel(
        _bag_body,
        out_type=jax.ShapeDtypeStruct((BATCH, DIM), jnp.float32),
        mesh=mesh,
        scratch_types=[
            pltpu.VMEM((NCH * SUB, IPS), jnp.int32),
            pltpu.VMEM((BPT * BAG,), jnp.float32),
            pltpu.VMEM((CB * BAG, DIM), jnp.float32),
            pltpu.VMEM((CB, DIM), jnp.float32),
            pltpu.SemaphoreType.DMA,
        ],
    )(idx3, scr2, weight)


def kernel(indices, scores, weight):
    idx3 = indices.astype(jnp.int32).reshape(NW, NCH * SUB, IPS)
    scr2 = scores.reshape(NW, BPT * BAG)
    return _bag_call(idx3, scr2, weight)


# speedup vs baseline: 2.4815x; 2.4815x over previous
import functools

import jax
import jax.numpy as jnp
from jax import lax
from jax.experimental import pallas as pl
from jax.experimental.pallas import tpu as pltpu
from jax.experimental.pallas import tpu_sc as plsc

SIZE = 1000000
DIM = 64
BATCH = 16384
BAG = 50

NCORE = 2
NSUB = 16
NW = NCORE * NSUB
BPT = BATCH // NW
CB = 8
NCH = BPT // CB
SUB = 4
IPS = CB * BAG // SUB
LANES = 16
DV = DIM // LANES


def _bag_body(idx_hbm, scr_hbm, tbl_hbm, out_hbm, idx_v, scr_v, rows_v, out_v,
              sem):
    wid = lax.axis_index("s") * NCORE + lax.axis_index("c")

    pltpu.sync_copy(idx_hbm.at[wid], idx_v)
    pltpu.sync_copy(scr_hbm.at[wid], scr_v.at[pl.ds(0, BPT * BAG)])

    def chunk(g, _):
        for s in range(SUB):
            pltpu.async_copy(
                tbl_hbm.at[idx_v.at[g * SUB + s]],
                rows_v.at[pl.ds(s * IPS, IPS)],
                sem,
            )
        for s in range(SUB):
            pltpu.make_async_copy(
                tbl_hbm.at[idx_v.at[g * SUB + s]],
                rows_v.at[pl.ds(s * IPS, IPS)],
                sem,
            ).wait()

        sbase = g * (CB * BAG)

        def bag(c, _):
            accs = [jnp.zeros((LANES,), jnp.float32) for _ in range(DV)]
            base = sbase + c * BAG
            for jj in range(0, BAG, LANES):
                svec = scr_v[pl.ds(base + jj, LANES)]
                for lane in range(min(LANES, BAG - jj)):
                    j = jj + lane
                    sc = svec[lane]
                    r = c * BAG + j
                    for t in range(DV):
                        accs[t] = accs[t] + sc * rows_v[r,
                                                        pl.ds(t * LANES, LANES)]
            for t in range(DV):
                out_v[c, pl.ds(t * LANES, LANES)] = accs[t]
            return 0

        lax.fori_loop(0, CB, bag, 0)
        pltpu.sync_copy(out_v, out_hbm.at[pl.ds(wid * BPT + g * CB, CB)])
        return 0

    lax.fori_loop(0, NCH, chunk, 0)


@jax.jit
def _bag_call(idx3, scr2, weight):
    mesh = plsc.VectorSubcoreMesh(core_axis_name="c", subcore_axis_name="s")
    return pl.kernel(
        _bag_body,
        out_type=jax.ShapeDtypeStruct((BATCH, DIM), jnp.float32),
        mesh=mesh,
        scratch_types=[
            pltpu.VMEM((NCH * SUB, IPS), jnp.int32),
            pltpu.VMEM((BPT * BAG + LANES,), jnp.float32),
            pltpu.VMEM((CB * BAG, DIM), jnp.float32),
            pltpu.VMEM((CB, DIM), jnp.float32),
            pltpu.SemaphoreType.DMA,
        ],
        compiler_params=pltpu.CompilerParams(use_tc_tiling_on_sc=False),
    )(idx3, scr2, weight)


def kernel(indices, scores, weight):
    idx3 = indices.astype(jnp.int32).reshape(NW, NCH * SUB, IPS)
    scr2 = scores.reshape(NW, BPT * BAG)
    return _bag_call(idx3, scr2, weight)


# speedup vs baseline: 2.7600x; 1.1122x over previous
import functools

import jax
import jax.numpy as jnp
from jax import lax
from jax.experimental import pallas as pl
from jax.experimental.pallas import tpu as pltpu
from jax.experimental.pallas import tpu_sc as plsc

SIZE = 1000000
DIM = 64
BATCH = 16384
BAG = 50

NCORE = 2
NSUB = 16
NW = NCORE * NSUB
BPT = BATCH // NW
CB = 8
NCH = BPT // CB
SUB = 4
IPS = CB * BAG // SUB
LANES = 16
DV = DIM // LANES


def _bag_body(idx_hbm, scr_hbm, tbl_hbm, out_hbm, idx_v, scr_v, rows_v, out_v,
              sem0, sem1):
    wid = lax.axis_index("s") * NCORE + lax.axis_index("c")
    sems = (sem0, sem1)

    pltpu.sync_copy(idx_hbm.at[wid], idx_v)
    pltpu.sync_copy(scr_hbm.at[wid], scr_v.at[pl.ds(0, BPT * BAG)])

    def issue(g, b):
        for s in range(SUB):
            pltpu.async_copy(
                tbl_hbm.at[idx_v.at[g * SUB + s]],
                rows_v.at[b, pl.ds(s * IPS, IPS)],
                sems[b],
            )

    def drain(g, b):
        for s in range(SUB):
            pltpu.make_async_copy(
                tbl_hbm.at[idx_v.at[g * SUB + s]],
                rows_v.at[b, pl.ds(s * IPS, IPS)],
                sems[b],
            ).wait()

    def compute(g, b):
        sbase = g * (CB * BAG)

        def bag(c, _):
            accs = [jnp.zeros((LANES,), jnp.float32) for _ in range(DV)]
            base = sbase + c * BAG
            for jj in range(0, BAG, LANES):
                svec = scr_v[pl.ds(base + jj, LANES)]
                for lane in range(min(LANES, BAG - jj)):
                    j = jj + lane
                    sc = svec[lane]
                    r = c * BAG + j
                    for t in range(DV):
                        accs[t] = accs[t] + sc * rows_v[b, r,
                                                        pl.ds(t * LANES, LANES)]
            for t in range(DV):
                out_v[c, pl.ds(t * LANES, LANES)] = accs[t]
            return 0

        lax.fori_loop(0, CB, bag, 0)
        pltpu.sync_copy(out_v, out_hbm.at[pl.ds(wid * BPT + g * CB, CB)])

    issue(0, 0)

    def pair(gg, _):
        g0 = 2 * gg
        issue(g0 + 1, 1)
        drain(g0, 0)
        compute(g0, 0)
        g1 = g0 + 1

        @pl.when(g1 + 1 < NCH)
        def _():
            issue(g1 + 1, 0)

        drain(g1, 1)
        compute(g1, 1)
        return 0

    lax.fori_loop(0, NCH // 2, pair, 0)


@jax.jit
def _bag_call(idx3, scr2, weight):
    mesh = plsc.VectorSubcoreMesh(core_axis_name="c", subcore_axis_name="s")
    return pl.kernel(
        _bag_body,
        out_type=jax.ShapeDtypeStruct((BATCH, DIM), jnp.float32),
        mesh=mesh,
        scratch_types=[
            pltpu.VMEM((NCH * SUB, IPS), jnp.int32),
            pltpu.VMEM((BPT * BAG + LANES,), jnp.float32),
            pltpu.VMEM((2, CB * BAG, DIM), jnp.float32),
            pltpu.VMEM((CB, DIM), jnp.float32),
            pltpu.SemaphoreType.DMA,
            pltpu.SemaphoreType.DMA,
        ],
        compiler_params=pltpu.CompilerParams(use_tc_tiling_on_sc=False),
    )(idx3, scr2, weight)


def kernel(indices, scores, weight):
    idx3 = indices.astype(jnp.int32).reshape(NW, NCH * SUB, IPS)
    scr2 = scores.reshape(NW, BPT * BAG)
    return _bag_call(idx3, scr2, weight)
